# Initial kernel scaffold; baseline (speedup 1.0000x reference)
import jax
import jax.numpy as jnp
from jax.experimental import pallas as pl


def kernel(x_user, x_item, edge_index_u2i, edge_index_i2u, edge_label_index, params):
    raise NotImplementedError("write your pallas kernel here")


# speedup vs baseline: 1.2998x; 1.2998x over previous
import jax
import jax.numpy as jnp
from jax import lax
from jax.experimental import pallas as pl
from jax.experimental.pallas import tpu as pltpu
from jax.experimental.pallas import tpu_sc as plsc

N = 10000
D = 256
EL = 4096
NC = 2
NS = 16
CHUNK = 128
ROWS_ACC = 10112
TPR = ROWS_ACC // NS
EPT = 10240
EPAD = NS * EPT
NCHUNK = EPT // CHUNK
RB = 1000


def _mm_body(x_ref, w_ref, yt_ref, r_ref):
    t = jnp.dot(x_ref[...], w_ref[...], preferred_element_type=jnp.float32)
    yt_ref[0] = t[:, :128]
    yt_ref[1] = t[:, 128:256]
    r_ref[...] = t[:, 256:]


def _mm(x, w):
    return pl.pallas_call(
        _mm_body,
        grid=(N // RB,),
        in_specs=[
            pl.BlockSpec((RB, D), lambda i: (i, 0)),
            pl.BlockSpec((D, 2 * D), lambda i: (0, 0)),
        ],
        out_specs=[
            pl.BlockSpec((2, RB, 128), lambda i: (0, i, 0)),
            pl.BlockSpec((RB, D), lambda i: (i, 0)),
        ],
        out_shape=[
            jax.ShapeDtypeStruct((2, N, 128), jnp.float32),
            jax.ShapeDtypeStruct((N, D), jnp.float32),
        ],
    )(x, w)


def _comb_body(s_ref, c_ref, r_ref, b_ref, h_ref):
    s = jnp.concatenate([s_ref[0], s_ref[1]], axis=1)
    cnt = jnp.concatenate([c_ref[0], c_ref[1]], axis=1)
    inv = 1.0 / jnp.maximum(cnt, 1.0)
    h_ref[...] = s * inv + r_ref[...] + b_ref[0:1, :]


def _combine(summed, cnt, r, b8):
    return pl.pallas_call(
        _comb_body,
        grid=(N // RB,),
        in_specs=[
            pl.BlockSpec((2, RB, 128), lambda i: (0, i, 0)),
            pl.BlockSpec((2, RB, 128), lambda i: (0, i, 0)),
            pl.BlockSpec((RB, D), lambda i: (i, 0)),
            pl.BlockSpec((8, D), lambda i: (0, 0)),
        ],
        out_specs=pl.BlockSpec((RB, D), lambda i: (i, 0)),
        out_shape=jax.ShapeDtypeStruct((N, D), jnp.float32),
    )(summed, cnt, r, b8)


def _agg_body(yt_hbm, src2_hbm, dst_hbm, zeros_hbm, out_hbm,
              idx_s, idx_d, rows, acc, sem):
    c = lax.axis_index("c")
    s = lax.axis_index("s")
    pltpu.sync_copy(zeros_hbm.at[pl.ds(s * TPR, TPR)],
                    acc.at[pl.ds(s * TPR, TPR)])
    plsc.subcore_barrier()

    def body(i, carry):
        base = s * EPT + i * CHUNK
        pltpu.sync_copy(src2_hbm.at[c, pl.ds(base, CHUNK)], idx_s)
        pltpu.sync_copy(dst_hbm.at[pl.ds(base, CHUNK)], idx_d)
        pltpu.async_copy(yt_hbm.at[idx_s], rows, sem).wait()
        pltpu.sync_copy(rows, acc.at[idx_d], add=True)
        return carry

    lax.fori_loop(0, NCHUNK, body, 0)
    plsc.subcore_barrier()
    pltpu.sync_copy(acc.at[pl.ds(s * TPR, TPR)],
                    out_hbm.at[c, pl.ds(s * TPR, TPR)])


def _agg(yt_flat, src2, dst_pad, zeros):
    mesh = plsc.VectorSubcoreMesh(core_axis_name="c", subcore_axis_name="s", num_cores=NC, num_subcores=NS)
    f = pl.kernel(
        _agg_body,
        out_type=jax.ShapeDtypeStruct((NC, ROWS_ACC, 128), jnp.float32),
        mesh=mesh,
        scratch_types=[
            pltpu.VMEM((CHUNK,), jnp.int32),
            pltpu.VMEM((CHUNK,), jnp.int32),
            pltpu.VMEM((CHUNK, 128), jnp.float32),
            pltpu.VMEM_SHARED((ROWS_ACC, 128), jnp.float32),
            pltpu.SemaphoreType.DMA,
        ],
    )
    return f(yt_flat, src2, dst_pad, zeros)


_ELT = EL // (NC * NS)


def _link_gather_body(hu_hbm, hi_hbm, el_hbm, gu_hbm, gi_hbm,
                      idx_u, idx_v, gu, gi, sem):
    c = lax.axis_index("c")
    s = lax.axis_index("s")
    wid = s * NC + c
    base = wid * _ELT
    pltpu.sync_copy(el_hbm.at[0, pl.ds(base, _ELT)], idx_u)
    pltpu.sync_copy(el_hbm.at[1, pl.ds(base, _ELT)], idx_v)
    pltpu.async_copy(hu_hbm.at[idx_u], gu, sem).wait()
    pltpu.async_copy(hi_hbm.at[idx_v], gi, sem).wait()
    pltpu.sync_copy(gu, gu_hbm.at[pl.ds(base, _ELT)])
    pltpu.sync_copy(gi, gi_hbm.at[pl.ds(base, _ELT)])


def _dot_body(gu_ref, gi_ref, o_ref):
    s = jnp.sum(gu_ref[...] * gi_ref[...], axis=1)
    o_ref[...] = jnp.broadcast_to(s[:, None], o_ref.shape)


def _link(hu, hi, el):
    mesh = plsc.VectorSubcoreMesh(core_axis_name="c", subcore_axis_name="s", num_cores=NC, num_subcores=NS)
    f = pl.kernel(
        _link_gather_body,
        out_type=[
            jax.ShapeDtypeStruct((EL, D), jnp.float32),
            jax.ShapeDtypeStruct((EL, D), jnp.float32),
        ],
        mesh=mesh,
        scratch_types=[
            pltpu.VMEM((_ELT,), jnp.int32),
            pltpu.VMEM((_ELT,), jnp.int32),
            pltpu.VMEM((_ELT, D), jnp.float32),
            pltpu.VMEM((_ELT, D), jnp.float32),
            pltpu.SemaphoreType.DMA,
        ],
    )
    gu, gi = f(hu, hi, el)
    pw = pl.pallas_call(
        _dot_body,
        grid=(EL // 512,),
        in_specs=[
            pl.BlockSpec((512, D), lambda i: (i, 0)),
            pl.BlockSpec((512, D), lambda i: (i, 0)),
        ],
        out_specs=pl.BlockSpec((512, 8), lambda i: (i, 0)),
        out_shape=jax.ShapeDtypeStruct((EL, 8), jnp.float32),
    )(gu, gi)
    return pw[:, 0]


def _prep_edges(ei):
    e = ei.shape[1]
    pad = EPAD - e
    src = jnp.concatenate([ei[0], jnp.zeros((pad,), jnp.int32)])
    dst = jnp.concatenate([ei[1], jnp.full((pad,), N, jnp.int32)])
    src2 = jnp.stack([src, src + N])
    return src2, dst


def kernel(x_user, x_item, edge_index_u2i, edge_index_i2u, edge_label_index, params):
    src2_ui, dst_ui = _prep_edges(edge_index_u2i)
    src2_iu, dst_iu = _prep_edges(edge_index_i2u)
    zeros = jnp.zeros((ROWS_ACC, 128), jnp.float32)
    ones_t = jnp.ones((2 * N, 128), jnp.float32)

    cnt_i = _agg(ones_t, src2_ui, dst_ui, zeros)
    cnt_u = _agg(ones_t, src2_iu, dst_iu, zeros)

    def layer(hu, hi, lp):
        wu = jnp.concatenate([lp["u2i"]["Wl"], lp["i2u"]["Wr"]], axis=1)
        wi = jnp.concatenate([lp["i2u"]["Wl"], lp["u2i"]["Wr"]], axis=1)
        ytu, ru = _mm(hu, wu)
        yti, ri = _mm(hi, wi)
        sum_i = _agg(ytu.reshape(2 * N, 128), src2_ui, dst_ui, zeros)
        sum_u = _agg(yti.reshape(2 * N, 128), src2_iu, dst_iu, zeros)
        b_ui = jnp.broadcast_to(lp["u2i"]["bl"], (8, D))
        b_iu = jnp.broadcast_to(lp["i2u"]["bl"], (8, D))
        hi_new = _combine(sum_i, cnt_i, ri, b_ui)
        hu_new = _combine(sum_u, cnt_u, ru, b_iu)
        return hu_new, hi_new

    hu, hi = x_user, x_item
    for name in ["enc0", "enc1"]:
        hu, hi = layer(hu, hi, params[name])
    fu, fi = hu, hi
    for name in ["dec0", "decL"]:
        fu, fi = layer(fu, fi, params[name])
    pred = _link(hu, hi, edge_label_index)
    return (hu, hi, fu, fi, pred)


# speedup vs baseline: 1.4608x; 1.1238x over previous
import jax
import jax.numpy as jnp
from jax import lax
from jax.experimental import pallas as pl
from jax.experimental.pallas import tpu as pltpu
from jax.experimental.pallas import tpu_sc as plsc

N = 10000
D = 256
EL = 4096
NC = 2
NS = 16
CHUNK = 128
ROWS_ACC = 10112
TPR = ROWS_ACC // NS
EPT = 10240
EPAD = NS * EPT
NCHUNK = EPT // CHUNK
NCHP = NCHUNK + 2
SPW = NCHP * CHUNK
RB = 1000


def _sc_mesh():
    return plsc.VectorSubcoreMesh(core_axis_name="c", subcore_axis_name="s",
                                  num_cores=NC, num_subcores=NS)


def _mm_body(x_ref, w_ref, yt_ref, r_ref):
    t = jnp.dot(x_ref[...], w_ref[...], preferred_element_type=jnp.float32)
    yt_ref[0] = t[:, :128]
    yt_ref[1] = t[:, 128:256]
    r_ref[...] = t[:, 256:]


def _mm(x, w):
    return pl.pallas_call(
        _mm_body,
        grid=(N // RB,),
        in_specs=[
            pl.BlockSpec((RB, D), lambda i: (i, 0)),
            pl.BlockSpec((D, 2 * D), lambda i: (0, 0)),
        ],
        out_specs=[
            pl.BlockSpec((2, RB, 128), lambda i: (0, i, 0)),
            pl.BlockSpec((RB, D), lambda i: (i, 0)),
        ],
        out_shape=[
            jax.ShapeDtypeStruct((2, N, 128), jnp.float32),
            jax.ShapeDtypeStruct((N, D), jnp.float32),
        ],
    )(x, w)


def _comb_body(s_ref, c_ref, r_ref, b_ref, h_ref):
    s = jnp.concatenate([s_ref[0], s_ref[1]], axis=1)
    inv = 1.0 / jnp.maximum(c_ref[:, 0:1], 1.0)
    h_ref[...] = s * inv + r_ref[...] + b_ref[0:1, :]


def _combine(summed, cnt, r, b8):
    return pl.pallas_call(
        _comb_body,
        grid=(N // RB,),
        in_specs=[
            pl.BlockSpec((2, RB, 128), lambda i: (0, i, 0)),
            pl.BlockSpec((RB, 128), lambda i: (i, 0)),
            pl.BlockSpec((RB, D), lambda i: (i, 0)),
            pl.BlockSpec((8, D), lambda i: (0, 0)),
        ],
        out_specs=pl.BlockSpec((RB, D), lambda i: (i, 0)),
        out_shape=jax.ShapeDtypeStruct((N, D), jnp.float32),
    )(summed, cnt, r, b8)


def _agg_body(yt_hbm, src2_hbm, dst_hbm, zeros_hbm, out_hbm,
              idx_s, dst_a, dst_b, rows_a, rows_b, acc,
              sem_ar, sem_ad, sem_br, sem_bd):
    c = lax.axis_index("c")
    s = lax.axis_index("s")
    pltpu.sync_copy(src2_hbm.at[c, s], idx_s)
    pltpu.sync_copy(zeros_hbm.at[pl.ds(s * TPR, TPR)],
                    acc.at[pl.ds(s * TPR, TPR)])
    plsc.subcore_barrier()
    base = s * SPW

    pltpu.async_copy(dst_hbm.at[pl.ds(base, CHUNK)], dst_a, sem_ad)
    pltpu.async_copy(dst_hbm.at[pl.ds(base + CHUNK, CHUNK)], dst_b, sem_bd)
    pltpu.async_copy(yt_hbm.at[idx_s.at[0]], rows_a, sem_ar)
    pltpu.async_copy(yt_hbm.at[idx_s.at[1]], rows_b, sem_br)

    def body(i, carry):
        pltpu.make_async_copy(zeros_hbm.at[pl.ds(0, CHUNK)], rows_a, sem_ar).wait()
        pltpu.make_async_copy(dst_hbm.at[pl.ds(0, CHUNK)], dst_a, sem_ad).wait()
        pltpu.sync_copy(rows_a, acc.at[dst_a], add=True)
        pltpu.async_copy(yt_hbm.at[idx_s.at[2 * i + 2]], rows_a, sem_ar)
        pltpu.async_copy(dst_hbm.at[pl.ds(base + (2 * i + 2) * CHUNK, CHUNK)],
                         dst_a, sem_ad)
        pltpu.make_async_copy(zeros_hbm.at[pl.ds(0, CHUNK)], rows_b, sem_br).wait()
        pltpu.make_async_copy(dst_hbm.at[pl.ds(0, CHUNK)], dst_b, sem_bd).wait()
        pltpu.sync_copy(rows_b, acc.at[dst_b], add=True)
        pltpu.async_copy(yt_hbm.at[idx_s.at[2 * i + 3]], rows_b, sem_br)
        pltpu.async_copy(dst_hbm.at[pl.ds(base + (2 * i + 3) * CHUNK, CHUNK)],
                         dst_b, sem_bd)
        return carry

    lax.fori_loop(0, NCHUNK // 2, body, 0)
    pltpu.make_async_copy(zeros_hbm.at[pl.ds(0, CHUNK)], rows_a, sem_ar).wait()
    pltpu.make_async_copy(zeros_hbm.at[pl.ds(0, CHUNK)], rows_b, sem_br).wait()
    pltpu.make_async_copy(dst_hbm.at[pl.ds(0, CHUNK)], dst_a, sem_ad).wait()
    pltpu.make_async_copy(dst_hbm.at[pl.ds(0, CHUNK)], dst_b, sem_bd).wait()
    plsc.subcore_barrier()
    pltpu.sync_copy(acc.at[pl.ds(s * TPR, TPR)],
                    out_hbm.at[c, pl.ds(s * TPR, TPR)])


def _agg(yt_flat, src2, dst3, zeros):
    f = pl.kernel(
        _agg_body,
        out_type=jax.ShapeDtypeStruct((NC, ROWS_ACC, 128), jnp.float32),
        mesh=_sc_mesh(),
        scratch_types=[
            pltpu.VMEM((NCHP, CHUNK), jnp.int32),
            pltpu.VMEM((CHUNK,), jnp.int32),
            pltpu.VMEM((CHUNK,), jnp.int32),
            pltpu.VMEM((CHUNK, 128), jnp.float32),
            pltpu.VMEM((CHUNK, 128), jnp.float32),
            pltpu.VMEM_SHARED((ROWS_ACC, 128), jnp.float32),
            pltpu.SemaphoreType.DMA,
            pltpu.SemaphoreType.DMA,
            pltpu.SemaphoreType.DMA,
            pltpu.SemaphoreType.DMA,
        ],
    )
    return f(yt_flat, src2, dst3, zeros)


def _cnt_body(dst_hbm, ones_hbm, zeros_hbm, out_hbm, idx_d, ones_v, acc):
    c = lax.axis_index("c")
    s = lax.axis_index("s")

    @pl.when(c == 0)
    def _():
        pltpu.sync_copy(ones_hbm, ones_v)
        pltpu.sync_copy(zeros_hbm.at[pl.ds(s * TPR, TPR)],
                        acc.at[pl.ds(s * TPR, TPR)])
        plsc.subcore_barrier()
        base = s * SPW

        def body(i, carry):
            pltpu.sync_copy(dst_hbm.at[pl.ds(base + i * CHUNK, CHUNK)], idx_d)
            pltpu.sync_copy(ones_v, acc.at[idx_d], add=True)
            return carry

        lax.fori_loop(0, NCHUNK, body, 0)
        plsc.subcore_barrier()
        pltpu.sync_copy(acc.at[pl.ds(s * TPR, TPR)],
                        out_hbm.at[pl.ds(s * TPR, TPR)])


def _cnt(dst_flat, ones128, zeros):
    f = pl.kernel(
        _cnt_body,
        out_type=jax.ShapeDtypeStruct((ROWS_ACC, 128), jnp.float32),
        mesh=_sc_mesh(),
        scratch_types=[
            pltpu.VMEM((CHUNK,), jnp.int32),
            pltpu.VMEM((CHUNK, 128), jnp.float32),
            pltpu.VMEM_SHARED((ROWS_ACC, 128), jnp.float32),
        ],
    )
    return f(dst_flat, ones128, zeros)


_ELT = EL // (NC * NS)


def _link_gather_body(hu_hbm, hi_hbm, el_hbm, gu_hbm, gi_hbm,
                      idx_u, idx_v, gu, gi, sem):
    c = lax.axis_index("c")
    s = lax.axis_index("s")
    wid = s * NC + c
    base = wid * _ELT
    pltpu.sync_copy(el_hbm.at[0, pl.ds(base, _ELT)], idx_u)
    pltpu.sync_copy(el_hbm.at[1, pl.ds(base, _ELT)], idx_v)
    pltpu.async_copy(hu_hbm.at[idx_u], gu, sem).wait()
    pltpu.async_copy(hi_hbm.at[idx_v], gi, sem).wait()
    pltpu.sync_copy(gu, gu_hbm.at[pl.ds(base, _ELT)])
    pltpu.sync_copy(gi, gi_hbm.at[pl.ds(base, _ELT)])


def _dot_body(gu_ref, gi_ref, o_ref):
    s = jnp.sum(gu_ref[...] * gi_ref[...], axis=1)
    o_ref[...] = jnp.broadcast_to(s[:, None], o_ref.shape)


def _link(hu, hi, el):
    f = pl.kernel(
        _link_gather_body,
        out_type=[
            jax.ShapeDtypeStruct((EL, D), jnp.float32),
            jax.ShapeDtypeStruct((EL, D), jnp.float32),
        ],
        mesh=_sc_mesh(),
        scratch_types=[
            pltpu.VMEM((_ELT,), jnp.int32),
            pltpu.VMEM((_ELT,), jnp.int32),
            pltpu.VMEM((_ELT, D), jnp.float32),
            pltpu.VMEM((_ELT, D), jnp.float32),
            pltpu.SemaphoreType.DMA,
        ],
    )
    gu, gi = f(hu, hi, el)
    pw = pl.pallas_call(
        _dot_body,
        grid=(EL // 512,),
        in_specs=[
            pl.BlockSpec((512, D), lambda i: (i, 0)),
            pl.BlockSpec((512, D), lambda i: (i, 0)),
        ],
        out_specs=pl.BlockSpec((512, 8), lambda i: (i, 0)),
        out_shape=jax.ShapeDtypeStruct((EL, 8), jnp.float32),
    )(gu, gi)
    return pw[:, 0]


def _prep_edges(ei):
    e = ei.shape[1]
    pad = EPAD - e
    src = jnp.concatenate([ei[0], jnp.zeros((pad,), jnp.int32)])
    dst = jnp.concatenate([ei[1], jnp.full((pad,), N, jnp.int32)])
    src3 = src.reshape(NS, NCHUNK, CHUNK)
    src3 = jnp.pad(src3, ((0, 0), (0, 2), (0, 0)))
    src2 = jnp.stack([src3, src3 + N]).reshape(NC, NS, NCHP, CHUNK)
    dst3 = dst.reshape(NS, NCHUNK, CHUNK)
    dst_flat = jnp.pad(dst3, ((0, 0), (0, 2), (0, 0)),
                       constant_values=N).reshape(NS * SPW)
    return src2, dst_flat


def kernel(x_user, x_item, edge_index_u2i, edge_index_i2u, edge_label_index, params):
    src2_ui, dst_ui = _prep_edges(edge_index_u2i)
    src2_iu, dst_iu = _prep_edges(edge_index_i2u)
    zeros = jnp.zeros((ROWS_ACC, 128), jnp.float32)
    ones128 = jnp.ones((CHUNK, 128), jnp.float32)

    cnt_i = _cnt(dst_ui, ones128, zeros)
    cnt_u = _cnt(dst_iu, ones128, zeros)

    def layer(hu, hi, lp):
        wu = jnp.concatenate([lp["u2i"]["Wl"], lp["i2u"]["Wr"]], axis=1)
        wi = jnp.concatenate([lp["i2u"]["Wl"], lp["u2i"]["Wr"]], axis=1)
        ytu, ru = _mm(hu, wu)
        yti, ri = _mm(hi, wi)
        sum_i = _agg(ytu.reshape(2 * N, 128), src2_ui, dst_ui, zeros)
        sum_u = _agg(yti.reshape(2 * N, 128), src2_iu, dst_iu, zeros)
        b_ui = jnp.broadcast_to(lp["u2i"]["bl"], (8, D))
        b_iu = jnp.broadcast_to(lp["i2u"]["bl"], (8, D))
        hi_new = _combine(sum_i, cnt_i, ri, b_ui)
        hu_new = _combine(sum_u, cnt_u, ru, b_iu)
        return hu_new, hi_new

    hu, hi = x_user, x_item
    for name in ["enc0", "enc1"]:
        hu, hi = layer(hu, hi, params[name])
    fu, fi = hu, hi
    for name in ["dec0", "decL"]:
        fu, fi = layer(fu, fi, params[name])
    pred = _link(hu, hi, edge_label_index)
    return (hu, hi, fu, fi, pred)
